# baseline (speedup 1.0000x reference)
import functools

import jax
import jax.numpy as jnp
from jax import lax
from jax.experimental import pallas as pl
from jax.experimental.pallas import tpu as pltpu
from jax.experimental.pallas import tpu_sc as plsc

NC, NS = 2, 16
NW = NC * NS
EC = 100


def _split2(v):
    hi = v.astype(jnp.bfloat16)
    lo = (v - hi.astype(jnp.float32)).astype(jnp.bfloat16)
    return hi, lo


def _tanh(x):
    x = jnp.clip(x, -4.97, 4.97)
    x2 = x * x
    num = x * (135135.0 + x2 * (17325.0 + x2 * (378.0 + x2)))
    den = 135135.0 + x2 * (62370.0 + x2 * (3150.0 + x2 * 28.0))
    return num / den


def _dotp(a, w):
    ahi, alo = _split2(a)
    whi, wlo = _split2(w)
    ap = jnp.concatenate([ahi, ahi, alo], axis=-1)
    wp = jnp.concatenate([whi, wlo, whi], axis=0)
    return jnp.dot(ap, wp, preferred_element_type=jnp.float32)


def _embed_body(x_ref, nb_ref, w0_ref, ea_ref, Wmh_ref, Wme_ref, bm_ref,
                h_ref, hw_ref, eb_ref, *, N, E, H, L, BBe):
    x2 = x_ref[...]
    h = _tanh(x2[:, :, None] * w0_ref[...][None, :, :] + nb_ref[...][None])
    h2 = h.reshape(BBe * N, H)
    h_ref[...] = h2
    hw = _dotp(h2, Wmh_ref[0])
    hw_ref[...] = jnp.concatenate([hw, jnp.zeros_like(hw)], axis=-1)
    ea = ea_ref[...]
    for l in range(L):
        eb_ref[l * E:(l + 1) * E, :] = _dotp(ea, Wme_ref[l]) + bm_ref[l:l + 1, :]


def _update_body(h_ref, agg_ref, dinv_ref, Wu_ref, bu_ref, g_ref, b_ref,
                 Wmh_ref, W1_ref, b1_ref, W2_ref, b2_ref, o1_ref, o2_ref,
                 *, l, L, H):
    h2 = h_ref[...]
    aggm = agg_ref[...] * dinv_ref[...]
    cat = jnp.concatenate([h2, aggm], axis=-1)
    u = _tanh(_dotp(cat, Wu_ref[l]) + bu_ref[l:l + 1, :])
    mu = jnp.mean(u, axis=-1, keepdims=True)
    var = jnp.mean((u - mu) ** 2, axis=-1, keepdims=True) + 1e-5
    r = jax.lax.rsqrt(var)
    r = r * (1.5 - 0.5 * var * r * r)
    hn = g_ref[l:l + 1, :] * (u - mu) * r + b_ref[l:l + 1, :]
    if l < L - 1:
        o1_ref[...] = hn
        hw = _dotp(hn, Wmh_ref[l + 1])
        o2_ref[...] = jnp.concatenate([hw, jnp.zeros_like(hw)], axis=-1)
    else:
        z = _tanh(_dotp(hn, W1_ref[...]) + b1_ref[...])
        pg = _dotp(z, W2_ref[...]) + b2_ref[...]
        o1_ref[...] = pg
        o2_ref[...] = pg


def _edge_body(hw_ref, idx_ref, dst_ref, eb_ref, zero_ref, agg_ref,
               idx_v, dst_v, eb_v, msg_v, agg_v, sem, *, N, E, H, BPW):
    wid = lax.axis_index("s") * NC + lax.axis_index("c")
    pltpu.sync_copy(dst_ref, dst_v)
    pltpu.sync_copy(eb_ref, eb_v)
    nchunk = E // EC

    def bloop(j, carry):
        b = wid * BPW + j
        pltpu.sync_copy(idx_ref.at[b], idx_v)
        pltpu.sync_copy(zero_ref, agg_v)

        def cloop(c, c2):
            pltpu.async_copy(hw_ref.at[idx_v.at[c]], msg_v, sem).wait()

            def ebody(e, c3):
                ge = c * EC + e
                d = dst_v[pl.ds(ge, 16)][0]
                d2 = lax.shift_right_logical(d, 1)
                doff = lax.bitwise_and(d, 1) * H
                e2 = lax.shift_right_logical(ge, 1)
                eoff = lax.bitwise_and(ge, 1) * H
                for k in range(H // 16):
                    xv = (msg_v[e, pl.ds(k * 16, 16)]
                          + eb_v[e2, pl.ds(eoff + k * 16, 16)])
                    plsc.addupdate(agg_v.at[d2, pl.ds(doff + k * 16, 16)],
                                   _tanh(xv))
                return c3

            lax.fori_loop(0, EC, ebody, 0)
            return c2

        lax.fori_loop(0, nchunk, cloop, 0)
        pltpu.sync_copy(agg_v, agg_ref.at[b])
        return carry

    lax.fori_loop(0, BPW, bloop, 0)


def _make_edge_call(B, N, E, H, BPW):
    mesh = plsc.VectorSubcoreMesh(core_axis_name="c", subcore_axis_name="s")
    body = functools.partial(_edge_body, N=N, E=E, H=H, BPW=BPW)
    return pl.kernel(
        body,
        mesh=mesh,
        out_type=jax.ShapeDtypeStruct((B, N // 2, 2 * H), jnp.float32),
        scratch_types=[
            pltpu.VMEM((E // EC, EC), jnp.int32),
            pltpu.VMEM((E + 16,), jnp.int32),
            pltpu.VMEM((E // 2, 2 * H), jnp.float32),
            pltpu.VMEM((EC, 2 * H), jnp.float32),
            pltpu.VMEM((N // 2, 2 * H), jnp.float32),
            pltpu.SemaphoreType.DMA,
        ],
    )


def kernel(x, edge_index, edge_attr, pg_min, pg_max, gen_bus_idx, gen_indices,
           W_emb, b_emb, Wm, bm, Wu, bu, gamma, beta, W1, b1, W2, b2):
    B, N = x.shape
    E = edge_index.shape[1]
    L, _, H = Wm.shape
    BPW = B // NW
    f32 = jnp.float32

    src = edge_index[0].astype(jnp.int32)
    dst = edge_index[1].astype(jnp.int32)
    dst_pad = jnp.pad(dst, (0, 16))
    deg = jnp.maximum(jnp.zeros((N,), f32).at[dst].add(1.0), 1.0)
    dinv_rows = jnp.tile(1.0 / deg, (B,))[:, None]
    idx_all = (src.reshape(E // EC, EC)[None, :, :]
               + (jnp.arange(B, dtype=jnp.int32) * N)[:, None, None])

    pmin = jnp.zeros((N,), x.dtype).at[gen_bus_idx].set(pg_min)
    pmax = jnp.zeros((N,), x.dtype).at[gen_bus_idx].set(pg_max)
    gmask = jnp.zeros((N,), x.dtype).at[gen_bus_idx].set(1.0)
    nbias = (pmin[:, None] * W_emb[1][None, :]
             + pmax[:, None] * W_emb[2][None, :]
             + gmask[:, None] * W_emb[3][None, :]
             + W_emb[4][None, :] + b_emb[None, :])
    w0 = W_emb[0][None, :]
    Wmh = Wm[:, :H, :]
    Wme = Wm[:, H:, :]
    zeros_nh = jnp.zeros((N // 2, 2 * H), f32)

    BBe = 64
    full = lambda s: pl.BlockSpec(s, lambda i: (0,) * len(s))
    rows_block = lambda rb: pl.BlockSpec((rb, H), lambda i: (i, 0))
    h0, hw0, eb_all = pl.pallas_call(
        functools.partial(_embed_body, N=N, E=E, H=H, L=L, BBe=BBe),
        grid=(B // BBe,),
        in_specs=[
            pl.BlockSpec((BBe, N), lambda i: (i, 0)),
            full((N, H)), full((1, H)), full((E, 2)),
            full((L, H, H)), full((L, 2, H)), full((L, H)),
        ],
        out_specs=[rows_block(BBe * N),
                   pl.BlockSpec((BBe * N, 2 * H), lambda i: (i, 0)),
                   pl.BlockSpec((L * E, H), lambda i: (0, 0))],
        out_shape=[jax.ShapeDtypeStruct((B * N, H), f32),
                   jax.ShapeDtypeStruct((B * N, 2 * H), f32),
                   jax.ShapeDtypeStruct((L * E, H), f32)],
    )(x, nbias, w0, edge_attr, Wmh, Wme, bm)

    edge_call = _make_edge_call(B, N, E, H, BPW)

    BBu = 32
    def update_call(l, h, agg):
        last = l == L - 1
        if last:
            out_shape = [jax.ShapeDtypeStruct((B * N, 1), f32),
                         jax.ShapeDtypeStruct((B * N, 1), f32)]
            obs = [pl.BlockSpec((BBu * N, 1), lambda i: (i, 0))] * 2
        else:
            out_shape = [jax.ShapeDtypeStruct((B * N, H), f32),
                         jax.ShapeDtypeStruct((B * N, 2 * H), f32)]
            obs = [rows_block(BBu * N),
                   pl.BlockSpec((BBu * N, 2 * H), lambda i: (i, 0))]
        return pl.pallas_call(
            functools.partial(_update_body, l=l, L=L, H=H),
            grid=(B // BBu,),
            in_specs=[
                rows_block(BBu * N), rows_block(BBu * N),
                pl.BlockSpec((BBu * N, 1), lambda i: (i, 0)),
                full((L, 2 * H, H)), full((L, H)), full((L, H)), full((L, H)),
                full((L, H, H)),
                full((H, H // 2)), full((1, H // 2)),
                full((H // 2, 1)), full((1, 1)),
            ],
            out_specs=obs,
            out_shape=out_shape,
        )(h, agg, dinv_rows, Wu, bu, gamma, beta, Wmh,
          W1, b1[None, :], W2, b2[None, :])

    eb2 = eb_all.reshape(L * E // 2, 2 * H)
    h, hw = h0, hw0
    EH = E // 2
    for l in range(L):
        agg = edge_call(hw, idx_all, dst_pad, eb2[l * EH:(l + 1) * EH],
                        zeros_nh)
        o1, o2 = update_call(l, h, agg.reshape(B * N, H))
        if l < L - 1:
            h, hw = o1, o2
        else:
            pg = o1

    return pg.reshape(B, N)[:, gen_indices]


# speedup vs baseline: 2.6706x; 2.6706x over previous
import functools

import jax
import jax.numpy as jnp
from jax import lax
from jax.experimental import pallas as pl
from jax.experimental.pallas import tpu as pltpu
from jax.experimental.pallas import tpu_sc as plsc

NC, NS = 2, 16
NW = NC * NS
EC = 100
BB = 32
BSC = 64


def _split2(v):
    hi = v.astype(jnp.bfloat16)
    lo = (v - hi.astype(jnp.float32)).astype(jnp.bfloat16)
    return hi, lo


def _tanh(x):
    x = jnp.clip(x, -6.0, 6.0)
    u = x * x
    num = x * (34459425.0 + u * (4729725.0 + u * (135135.0 + u * (990.0 + u))))
    den = 34459425.0 + u * (16216200.0 + u * (945945.0
                                              + u * (13860.0 + u * 45.0)))
    return num / den


def _dotp(a, w):
    ahi, alo = _split2(a)
    whi, wlo = _split2(w)
    ap = jnp.concatenate([ahi, ahi, alo], axis=-1)
    wp = jnp.concatenate([whi, wlo, whi], axis=0)
    return jnp.dot(ap, wp, preferred_element_type=jnp.float32)


def _dotbf(a, w):
    return jnp.dot(a.astype(jnp.bfloat16), w.astype(jnp.bfloat16),
                   preferred_element_type=jnp.float32)


def _bfr(v):
    return v.astype(jnp.bfloat16).astype(jnp.float32)


def _layer_norm_rows(u, g, b):
    mu = jnp.mean(u, axis=-1, keepdims=True)
    var = jnp.mean((u - mu) ** 2, axis=-1, keepdims=True) + 1e-5
    r = jax.lax.rsqrt(var)
    r = r * (1.5 - 0.5 * var * r * r)
    return g * (u - mu) * r + b


def _gnn_body(xC_ref, A_ref, M_ref, deginv_ref, nbias_ref, w0_ref, ea_ref,
              Wmh_ref, Wmet_ref, bmt_ref, Wu_ref, bu_ref, g_ref, b_ref,
              W1_ref, b1_ref, W2_ref, b2_ref, out_ref, scrB, scrA,
              *, Np, E, H, L):
    f32 = jnp.float32

    x2 = xC_ref[0]
    nb = nbias_ref[...]
    w0 = w0_ref[...]
    h = _tanh(_bfr(x2)[:, :, None] * _bfr(w0)[None, :, :] + nb[None, :, :])
    h2 = h.reshape(BB * Np, H)

    A = A_ref[...]
    M = M_ref[...]
    deginv = deginv_ref[...]
    ea = ea_ref[...]

    for l in range(L):
        hW = _dotbf(h2, Wmh_ref[l])
        for b in range(BB):
            scrB[:, b * H:(b + 1) * H] = jax.lax.slice(
                hW, (b * Np, 0), ((b + 1) * Np, H))
        hi, lo = _split2(scrB[...])
        t = (jnp.dot(A, hi, preferred_element_type=f32)
             + jnp.dot(A, lo, preferred_element_type=f32))
        ebig = _dotbf(ea, Wmet_ref[l]) + bmt_ref[l:l + 1, :]
        msg = _tanh(t + ebig)
        mhi, mlo = _split2(msg)
        agg = (jnp.dot(M, mhi, preferred_element_type=f32)
               + jnp.dot(M, mlo, preferred_element_type=f32)) * deginv
        for b in range(BB):
            scrA[b * Np:(b + 1) * Np, :] = jax.lax.slice(
                agg, (0, b * H), (Np, (b + 1) * H))
        cat = jnp.concatenate([h2, scrA[...]], axis=-1)
        u = _tanh(_dotbf(cat, Wu_ref[l]) + bu_ref[l:l + 1, :])
        h2 = _layer_norm_rows(u, g_ref[l:l + 1, :], b_ref[l:l + 1, :])

    z = _tanh(_dotbf(h2, W1_ref[...]) + b1_ref[...])
    pg = _dotbf(z, W2_ref[...]) + b2_ref[...]
    out_ref[0] = pg


def _embed_body(x_ref, nb_ref, w0_ref, ea_ref, Wmh_ref, Wme_ref, bm_ref,
                h_ref, hw_ref, eb_ref, *, N, E, H, L, BBe):
    x2 = x_ref[...]
    h = _tanh(_bfr(x2)[:, :, None] * _bfr(w0_ref[...])[None, :, :]
              + nb_ref[...][None])
    h2 = h.reshape(BBe * N, H)
    h_ref[...] = h2
    hw = _dotbf(h2, Wmh_ref[0])
    hw_ref[...] = jnp.concatenate([hw, jnp.zeros_like(hw)], axis=-1)
    ea = ea_ref[...]
    for l in range(L):
        eb_ref[l * E:(l + 1) * E, :] = _dotbf(ea, Wme_ref[l]) + bm_ref[l:l + 1, :]


def _update_body(h_ref, agg_ref, dinv_ref, Wu_ref, bu_ref, g_ref, b_ref,
                 Wmh_ref, W1_ref, b1_ref, W2_ref, b2_ref, o1_ref, o2_ref,
                 *, l, L, H):
    h2 = h_ref[...]
    aggm = agg_ref[...] * dinv_ref[...]
    cat = jnp.concatenate([h2, aggm], axis=-1)
    u = _tanh(_dotbf(cat, Wu_ref[l]) + bu_ref[l:l + 1, :])
    hn = _layer_norm_rows(u, g_ref[l:l + 1, :], b_ref[l:l + 1, :])
    if l < L - 1:
        o1_ref[...] = hn
        hw = _dotbf(hn, Wmh_ref[l + 1])
        o2_ref[...] = jnp.concatenate([hw, jnp.zeros_like(hw)], axis=-1)
    else:
        z = _tanh(_dotbf(hn, W1_ref[...]) + b1_ref[...])
        pg = _dotbf(z, W2_ref[...]) + b2_ref[...]
        o1_ref[...] = pg
        o2_ref[...] = pg


def _edge_body(hw_ref, idx_ref, dst_ref, eb_ref, zero_ref, agg_ref,
               idx_v, dst_v, eb_v, msg_v, agg_v, sem, *, N, E, H, BPW):
    wid = lax.axis_index("s") * NC + lax.axis_index("c")
    pltpu.sync_copy(dst_ref, dst_v)
    pltpu.sync_copy(eb_ref, eb_v)
    nchunk = E // EC

    def bloop(j, carry):
        b = wid * BPW + j
        pltpu.sync_copy(idx_ref.at[b], idx_v)
        pltpu.sync_copy(zero_ref, agg_v)
        for c in range(nchunk):
            pltpu.async_copy(hw_ref.at[idx_v.at[c]], msg_v, sem).wait()
            base = c * EC

            def ebody(p, c3, base=base):
                e0 = 2 * p
                ge = base + e0
                er = lax.shift_right_logical(ge, 1)
                d0 = dst_v[pl.ds(ge, 16)][0]
                d1 = dst_v[pl.ds(ge + 1, 16)][0]
                d0r = lax.shift_right_logical(d0, 1)
                d0o = lax.bitwise_and(d0, 1) * H
                d1r = lax.shift_right_logical(d1, 1)
                d1o = lax.bitwise_and(d1, 1) * H
                for k in range(H // 16):
                    x0 = (msg_v[e0, pl.ds(k * 16, 16)]
                          + eb_v[er, pl.ds(k * 16, 16)])
                    x1 = (msg_v[e0 + 1, pl.ds(k * 16, 16)]
                          + eb_v[er, pl.ds(H + k * 16, 16)])
                    plsc.addupdate(agg_v.at[d0r, pl.ds(d0o + k * 16, 16)],
                                   _tanh(x0))
                    plsc.addupdate(agg_v.at[d1r, pl.ds(d1o + k * 16, 16)],
                                   _tanh(x1))
                return c3

            lax.fori_loop(0, EC // 2, ebody, 0)
        pltpu.sync_copy(agg_v, agg_ref.at[b])
        return carry

    lax.fori_loop(0, BPW, bloop, 0)


def _make_edge_call(Bs, N, E, H, BPW):
    mesh = plsc.VectorSubcoreMesh(core_axis_name="c", subcore_axis_name="s")
    body = functools.partial(_edge_body, N=N, E=E, H=H, BPW=BPW)
    return pl.kernel(
        body,
        mesh=mesh,
        out_type=jax.ShapeDtypeStruct((Bs, N // 2, 2 * H), jnp.float32),
        scratch_types=[
            pltpu.VMEM((E // EC, EC), jnp.int32),
            pltpu.VMEM((E + 16,), jnp.int32),
            pltpu.VMEM((E // 2, 2 * H), jnp.float32),
            pltpu.VMEM((EC, 2 * H), jnp.float32),
            pltpu.VMEM((N // 2, 2 * H), jnp.float32),
            pltpu.SemaphoreType.DMA,
        ],
    )


def kernel(x, edge_index, edge_attr, pg_min, pg_max, gen_bus_idx, gen_indices,
           W_emb, b_emb, Wm, bm, Wu, bu, gamma, beta, W1, b1, W2, b2):
    B, N = x.shape
    E = edge_index.shape[1]
    L, _, H = Wm.shape
    f32 = jnp.float32
    bf16 = jnp.bfloat16
    BTC = B - BSC
    BPW = BSC // NW
    Np = ((N + 7) // 8) * 8

    src = edge_index[0].astype(jnp.int32)
    dst = edge_index[1].astype(jnp.int32)
    deg = jnp.maximum(jnp.zeros((N,), f32).at[dst].add(1.0), 1.0)

    pmin = jnp.zeros((Np,), x.dtype).at[gen_bus_idx].set(pg_min)
    pmax = jnp.zeros((Np,), x.dtype).at[gen_bus_idx].set(pg_max)
    gmask = jnp.zeros((Np,), x.dtype).at[gen_bus_idx].set(1.0)
    bfr = lambda v: v.astype(jnp.bfloat16).astype(f32)
    nbias_p = (bfr(pmin)[:, None] * bfr(W_emb[1])[None, :]
               + bfr(pmax)[:, None] * bfr(W_emb[2])[None, :]
               + bfr(gmask)[:, None] * bfr(W_emb[3])[None, :]
               + bfr(W_emb[4])[None, :] + b_emb[None, :])
    nbias = nbias_p[:N]
    w0 = W_emb[0][None, :]
    Wmh = Wm[:, :H, :]
    Wme = Wm[:, H:, :]
    full = lambda s: pl.BlockSpec(s, lambda i: (0,) * len(s))

    nids = jnp.arange(Np, dtype=src.dtype)
    A = (src[:, None] == nids[None, :]).astype(bf16)
    M = (dst[None, :] == nids[:, None]).astype(bf16)
    degp = jnp.maximum(jnp.zeros((Np,), f32).at[dst].add(1.0), 1.0)
    deginv = (1.0 / degp)[:, None]
    xC = jnp.pad(x[:BTC], ((0, 0), (0, Np - N))).reshape(BTC // BB, BB, Np)
    Wmet = jnp.tile(Wme, (1, 1, BB))
    bmt = jnp.tile(bm, (1, BB))

    pg_tc = pl.pallas_call(
        functools.partial(_gnn_body, Np=Np, E=E, H=H, L=L),
        grid=(BTC // BB,),
        in_specs=[
            pl.BlockSpec((1, BB, Np), lambda i: (i, 0, 0)),
            full((E, Np)), full((Np, E)), full((Np, 1)), full((Np, H)),
            full((1, H)), full((E, 2)),
            full((L, H, H)), full((L, 2, BB * H)), full((L, BB * H)),
            full((L, 2 * H, H)), full((L, H)), full((L, H)), full((L, H)),
            full((H, H // 2)), full((1, H // 2)),
            full((H // 2, 1)), full((1, 1)),
        ],
        out_specs=pl.BlockSpec((1, BB * Np, 1), lambda i: (i, 0, 0)),
        out_shape=jax.ShapeDtypeStruct((BTC // BB, BB * Np, 1), f32),
        scratch_shapes=[
            pltpu.VMEM((Np, BB * H), f32),
            pltpu.VMEM((BB * Np, H), f32),
        ],
    )(xC, A, M, deginv, nbias_p, w0, edge_attr,
      Wmh, Wmet, bmt, Wu, bu, gamma, beta,
      W1, b1[None, :], W2, b2[None, :]).reshape(BTC, Np)

    xs = x[BTC:]
    dst_pad = jnp.pad(dst, (0, 16))
    dinv_rows = jnp.tile(1.0 / deg, (BSC,))[:, None]
    idx_all = (src.reshape(E // EC, EC)[None, :, :]
               + (jnp.arange(BSC, dtype=jnp.int32) * N)[:, None, None])
    zeros_nh = jnp.zeros((N // 2, 2 * H), f32)

    BBe = BSC
    rows_block = lambda rb: pl.BlockSpec((rb, H), lambda i: (i, 0))
    h0, hw0, eb_all = pl.pallas_call(
        functools.partial(_embed_body, N=N, E=E, H=H, L=L, BBe=BBe),
        grid=(BSC // BBe,),
        in_specs=[
            pl.BlockSpec((BBe, N), lambda i: (i, 0)),
            full((N, H)), full((1, H)), full((E, 2)),
            full((L, H, H)), full((L, 2, H)), full((L, H)),
        ],
        out_specs=[rows_block(BBe * N),
                   pl.BlockSpec((BBe * N, 2 * H), lambda i: (i, 0)),
                   pl.BlockSpec((L * E, H), lambda i: (0, 0))],
        out_shape=[jax.ShapeDtypeStruct((BSC * N, H), f32),
                   jax.ShapeDtypeStruct((BSC * N, 2 * H), f32),
                   jax.ShapeDtypeStruct((L * E, H), f32)],
    )(xs, nbias, w0, edge_attr, Wmh, Wme, bm)

    edge_call = _make_edge_call(BSC, N, E, H, BPW)

    BBu = 32
    def update_call(l, h, agg):
        last = l == L - 1
        if last:
            out_shape = [jax.ShapeDtypeStruct((BSC * N, 1), f32)] * 2
            obs = [pl.BlockSpec((BBu * N, 1), lambda i: (i, 0))] * 2
        else:
            out_shape = [jax.ShapeDtypeStruct((BSC * N, H), f32),
                         jax.ShapeDtypeStruct((BSC * N, 2 * H), f32)]
            obs = [rows_block(BBu * N),
                   pl.BlockSpec((BBu * N, 2 * H), lambda i: (i, 0))]
        return pl.pallas_call(
            functools.partial(_update_body, l=l, L=L, H=H),
            grid=(BSC // BBu,),
            in_specs=[
                rows_block(BBu * N), rows_block(BBu * N),
                pl.BlockSpec((BBu * N, 1), lambda i: (i, 0)),
                full((L, 2 * H, H)), full((L, H)), full((L, H)), full((L, H)),
                full((L, H, H)),
                full((H, H // 2)), full((1, H // 2)),
                full((H // 2, 1)), full((1, 1)),
            ],
            out_specs=obs,
            out_shape=out_shape,
        )(h, agg, dinv_rows, Wu, bu, gamma, beta, Wmh,
          W1, b1[None, :], W2, b2[None, :])

    eb2 = eb_all.reshape(L * E // 2, 2 * H)
    h, hw = h0, hw0
    EH = E // 2
    for l in range(L):
        agg = edge_call(hw, idx_all, dst_pad, eb2[l * EH:(l + 1) * EH],
                        zeros_nh)
        o1, o2 = update_call(l, h, agg.reshape(BSC * N, H))
        if l < L - 1:
            h, hw = o1, o2
        else:
            pg_sc = o1.reshape(BSC, N)

    pg_bn = jnp.concatenate([pg_tc[:, :N], pg_sc], axis=0)
    return pg_bn[:, gen_indices]


# speedup vs baseline: 3.8693x; 1.4488x over previous
import functools

import jax
import jax.numpy as jnp
from jax.experimental import pallas as pl
from jax.experimental.pallas import tpu as pltpu

BB = 32


def _split2(v):
    hi = v.astype(jnp.bfloat16)
    lo = (v - hi.astype(jnp.float32)).astype(jnp.bfloat16)
    return hi, lo


def _tanh(x):
    x = jnp.clip(x, -4.97, 4.97)
    x2 = x * x
    num = x * (135135.0 + x2 * (17325.0 + x2 * (378.0 + x2)))
    den = 135135.0 + x2 * (62370.0 + x2 * (3150.0 + x2 * 28.0))
    return num / den


def _dotp(a, w):
    ahi, alo = _split2(a)
    whi, wlo = _split2(w)
    ap = jnp.concatenate([ahi, ahi, alo], axis=-1)
    wp = jnp.concatenate([whi, wlo, whi], axis=0)
    return jnp.dot(ap, wp, preferred_element_type=jnp.float32)


def _dotbf(a, w):
    return jnp.dot(a.astype(jnp.bfloat16), w.astype(jnp.bfloat16),
                   preferred_element_type=jnp.float32)


def _bfr(v):
    return v.astype(jnp.bfloat16).astype(jnp.float32)


def _gnn_body(xC_ref, A_ref, M_ref, deginv_ref, nbias_ref, w0_ref, ea_ref,
              Wmh_ref, Wmet_ref, bmt_ref, Wu_ref, bu_ref, g_ref, b_ref,
              W1_ref, b1_ref, W2_ref, b2_ref, out_ref, scrB, scrA,
              *, Np, E, H, L):
    f32 = jnp.float32

    x2 = xC_ref[0]
    nb = nbias_ref[...]
    w0 = w0_ref[...]
    h = _tanh(_bfr(x2)[:, :, None] * _bfr(w0)[None, :, :] + nb[None, :, :])
    h2 = h.reshape(BB * Np, H)

    A = A_ref[...]
    M = M_ref[...]
    deginv = deginv_ref[...]
    ea = ea_ref[...]

    for l in range(L):
        hW = _dotbf(h2, Wmh_ref[l])
        for b in range(BB):
            scrB[:, b * H:(b + 1) * H] = jax.lax.slice(
                hW, (b * Np, 0), ((b + 1) * Np, H))
        hWn = scrB[...]
        hi, lo = _split2(hWn)
        t = (jnp.dot(A, hi, preferred_element_type=f32)
             + jnp.dot(A, lo, preferred_element_type=f32))
        ebig = _dotbf(ea, Wmet_ref[l]) + bmt_ref[l:l + 1, :]
        msg = _tanh(t + ebig)
        mhi, mlo = _split2(msg)
        agg = (jnp.dot(M, mhi, preferred_element_type=f32)
               + jnp.dot(M, mlo, preferred_element_type=f32)) * deginv
        for b in range(BB):
            scrA[b * Np:(b + 1) * Np, :] = jax.lax.slice(
                agg, (0, b * H), (Np, (b + 1) * H))
        agg2 = scrA[...]
        cat = jnp.concatenate([h2, agg2], axis=-1)
        u = _tanh(_dotbf(cat, Wu_ref[l]) + bu_ref[l:l + 1, :])
        mu = jnp.mean(u, axis=-1, keepdims=True)
        var = jnp.mean((u - mu) ** 2, axis=-1, keepdims=True) + 1e-5
        r = jax.lax.rsqrt(var)
        r = r * (1.5 - 0.5 * var * r * r)
        h2 = g_ref[l:l + 1, :] * (u - mu) * r + b_ref[l:l + 1, :]

    z = _tanh(_dotbf(h2, W1_ref[...]) + b1_ref[...])
    pg = _dotbf(z, W2_ref[...]) + b2_ref[...]
    out_ref[0] = pg


def kernel(x, edge_index, edge_attr, pg_min, pg_max, gen_bus_idx, gen_indices,
           W_emb, b_emb, Wm, bm, Wu, bu, gamma, beta, W1, b1, W2, b2):
    B, N = x.shape
    E = edge_index.shape[1]
    L, _, H = Wm.shape
    Np = ((N + 7) // 8) * 8
    bf16 = jnp.bfloat16

    src = edge_index[0]
    dst = edge_index[1]
    nids = jnp.arange(Np, dtype=src.dtype)
    A = (src[:, None] == nids[None, :]).astype(bf16)
    M = (dst[None, :] == nids[:, None]).astype(bf16)
    deg = jnp.maximum(jnp.zeros((Np,), jnp.float32).at[dst].add(1.0), 1.0)
    deginv = (1.0 / deg)[:, None]

    pmin = jnp.zeros((Np,), x.dtype).at[gen_bus_idx].set(pg_min)
    pmax = jnp.zeros((Np,), x.dtype).at[gen_bus_idx].set(pg_max)
    gmask = jnp.zeros((Np,), x.dtype).at[gen_bus_idx].set(1.0)
    bfr = lambda v: v.astype(bf16).astype(jnp.float32)
    nbias = (bfr(pmin)[:, None] * bfr(W_emb[1])[None, :]
             + bfr(pmax)[:, None] * bfr(W_emb[2])[None, :]
             + bfr(gmask)[:, None] * bfr(W_emb[3])[None, :]
             + bfr(W_emb[4])[None, :] + b_emb[None, :])
    w0 = W_emb[0][None, :]

    xC = jnp.pad(x, ((0, 0), (0, Np - N))).reshape(B // BB, BB, Np)
    Wmh = Wm[:, :H, :]
    Wmet = jnp.tile(Wm[:, H:, :], (1, 1, BB))
    bmt = jnp.tile(bm, (1, BB))

    grid = (B // BB,)
    full = lambda s: pl.BlockSpec(s, lambda i: (0,) * len(s))
    body = functools.partial(_gnn_body, Np=Np, E=E, H=H, L=L)
    pg_full = pl.pallas_call(
        body,
        grid=grid,
        in_specs=[
            pl.BlockSpec((1, BB, Np), lambda i: (i, 0, 0)),
            full((E, Np)), full((Np, E)), full((Np, 1)), full((Np, H)),
            full((1, H)), full((E, 2)),
            full((L, H, H)), full((L, 2, BB * H)), full((L, BB * H)),
            full((L, 2 * H, H)), full((L, H)), full((L, H)), full((L, H)),
            full((H, H // 2)), full((1, H // 2)),
            full((H // 2, 1)), full((1, 1)),
        ],
        out_specs=pl.BlockSpec((1, BB * Np, 1), lambda i: (i, 0, 0)),
        out_shape=jax.ShapeDtypeStruct((B // BB, BB * Np, 1), jnp.float32),
        scratch_shapes=[
            pltpu.VMEM((Np, BB * H), jnp.float32),
            pltpu.VMEM((BB * Np, H), jnp.float32),
        ],
    )(xC, A, M, deginv, nbias, w0, edge_attr,
      Wmh, Wmet, bmt, Wu, bu, gamma, beta,
      W1, b1[None, :], W2, b2[None, :])

    pg_bn = pg_full.reshape(B, Np)
    return pg_bn[:, gen_indices]
